# baseline (speedup 1.0000x reference)
import functools

import jax
import jax.numpy as jnp
from jax import lax
from jax.experimental import pallas as pl
from jax.experimental.pallas import tpu as pltpu
from jax.experimental.pallas import tpu_sc as plsc

_DIM = 768
_H = 12
_DH = 64
_E = 6
_S = 2048

_SB = 256
_SQ = 512

_bf16 = jnp.bfloat16
_f32 = jnp.float32


def _head_expand_matrix():
  rows = jax.lax.broadcasted_iota(jnp.int32, (_H, _DIM), 0)
  cols = jax.lax.broadcasted_iota(jnp.int32, (_H, _DIM), 1)
  return (cols // _DH == rows).astype(_f32)


def _top1_sigmoid(scores_em):
  best_s = jnp.full((scores_em.shape[0], _H), -jnp.inf, _f32)
  best_i = jnp.zeros((scores_em.shape[0], _H), jnp.int32)
  for e in range(_E):
    se = scores_em[:, e * _H:(e + 1) * _H]
    upd = se > best_s
    best_s = jnp.where(upd, se, best_s)
    best_i = jnp.where(upd, e, best_i)
  return jax.nn.sigmoid(best_s), best_i


def _layernorm(xb, g, b):
  m = jnp.mean(xb, axis=-1, keepdims=True)
  xc = xb - m
  v = jnp.mean(xc * xc, axis=-1, keepdims=True)
  return xc / jnp.sqrt(v + 1e-5) * g + b


def _pre_body(x_ref, g_ref, b_ref, wq_ref, wk_ref, wsv_ref, wso_ref, wv_ref,
              q_ref, k_ref, v_ref, mow_ref, moi_ref):
  xn = _layernorm(x_ref[...], g_ref[...], b_ref[...])
  xnb = xn.astype(_bf16)
  q_ref[...] = jnp.dot(xnb, wq_ref[...], preferred_element_type=_f32).astype(_bf16)
  k_ref[...] = jnp.dot(xnb, wk_ref[...], preferred_element_type=_f32).astype(_bf16)
  sv = jnp.dot(xnb, wsv_ref[...], preferred_element_type=_f32)
  so = jnp.dot(xnb, wso_ref[...], preferred_element_type=_f32)
  vw, vi = _top1_sigmoid(sv)
  ow, oi = _top1_sigmoid(so)
  mow_ref[...] = ow
  moi_ref[...] = oi
  rexp = _head_expand_matrix()
  vw_b = vw.astype(_bf16).astype(_f32)
  vacc = jnp.zeros((x_ref.shape[0], _DIM), _f32)
  for e in range(_E):
    ve = jnp.dot(xnb, wv_ref[e], preferred_element_type=_f32)
    ve_b = ve.astype(_bf16).astype(_f32)
    m768 = jnp.dot(jnp.where(vi == e, vw_b, 0.0), rexp,
                   preferred_element_type=_f32)
    vacc += m768 * ve_b
  v_ref[...] = vacc.astype(_bf16)


def _attn_body(q_ref, k_ref, v_ref, o_ref):
  scale = 1.0 / jnp.sqrt(jnp.float32(_DH))
  for h in range(_H):
    sl = slice(h * _DH, (h + 1) * _DH)
    qh = q_ref[:, sl]
    kh = k_ref[:, sl]
    vh = v_ref[:, sl]
    s = jax.lax.dot_general(qh, kh, (((1,), (1,)), ((), ())),
                            preferred_element_type=_f32) * scale
    m = jnp.max(s, axis=-1, keepdims=True)
    p = jnp.exp(s - m)
    l = jnp.sum(p, axis=-1, keepdims=True)
    pn = (p * (1.0 / l)).astype(_bf16)
    o = jax.lax.dot_general(pn, vh, (((1,), (0,)), ((), ())),
                            preferred_element_type=_f32)
    o_ref[:, sl] = o.astype(_bf16)


def _post_body(x_ref, o_ref, mow_ref, moi_ref, wo_ref, g2_ref, b2_ref,
               wg_ref, x1_ref, xw_ref, wv_out_ref, eid_ref):
  rexp = _head_expand_matrix()
  of = o_ref[...].astype(_f32)
  ow_b = mow_ref[...].astype(_bf16).astype(_f32)
  oi = moi_ref[...]
  acc = jnp.zeros((x_ref.shape[0], _DIM), _f32)
  for e in range(_E):
    m768 = jnp.dot(jnp.where(oi == e, ow_b, 0.0), rexp,
                   preferred_element_type=_f32)
    oe = (of * m768).astype(_bf16)
    acc += jnp.dot(oe, wo_ref[e], preferred_element_type=_f32)
  x1 = x_ref[...] + acc
  xn2 = _layernorm(x1, g2_ref[...], b2_ref[...])
  xn2b = xn2.astype(_bf16)
  logits = jnp.dot(xn2b, wg_ref[...], preferred_element_type=_f32)
  lm = jnp.max(logits, axis=-1, keepdims=True)
  ex = jnp.exp(logits - lm)
  gw = 1.0 / jnp.sum(ex, axis=-1, keepdims=True)
  gb = jnp.full((x_ref.shape[0], 1), -jnp.inf, _f32)
  gi = jnp.zeros((x_ref.shape[0], 1), jnp.int32)
  for e in range(_E):
    ge = logits[:, e:e + 1]
    upd = ge > gb
    gb = jnp.where(upd, ge, gb)
    gi = jnp.where(upd, e, gi)
  xn2_b32 = xn2b.astype(_f32)
  gw_b = gw.astype(_bf16).astype(_f32)
  x1_ref[...] = x1
  xw_ref[...] = (xn2_b32 * gw_b).astype(_bf16)
  wv_out_ref[...] = gw_b
  eid_ref[...] = gi


def _const_spec(shape):
  nd = len(shape)
  return pl.BlockSpec(shape, lambda *_: (0,) * nd)


_BLK = 256
_PAD_S = _S + _E * _BLK
_NB = _PAD_S // _BLK
_NSC = 32


def _route_prep_body(eid_full_ref, eid_blk_ref, dest_ref, bexp_ref):
  i = pl.program_id(0)
  e_iota_full = jax.lax.broadcasted_iota(jnp.int32, (_S, _E), 1)
  oh_full = (eid_full_ref[...] == e_iota_full).astype(_f32)
  counts = jnp.sum(oh_full, axis=0, keepdims=True)
  pc = jnp.floor((counts + (_BLK - 1)) / _BLK) * _BLK
  row_iota = jax.lax.broadcasted_iota(jnp.int32, (_S, _E), 0)
  prior = jnp.sum(jnp.where(row_iota < i * _SB, oh_full, 0.0),
                  axis=0, keepdims=True)
  e_iota_blk = jax.lax.broadcasted_iota(jnp.int32, (_SB, _E), 1)
  oh_blk = (eid_blk_ref[...] == e_iota_blk).astype(_f32)
  lt_r = jax.lax.broadcasted_iota(jnp.int32, (_SB, _SB), 0)
  lt_c = jax.lax.broadcasted_iota(jnp.int32, (_SB, _SB), 1)
  lt = (lt_c <= lt_r).astype(_f32)
  rank = jnp.dot(lt, oh_blk, preferred_element_type=_f32,
                 precision=jax.lax.Precision.HIGHEST)
  dest = jnp.zeros((_SB, 1), _f32)
  poff_e = jnp.zeros((1, 1), _f32)
  ends = []
  for e in range(_E):
    seg = poff_e + prior[:, e:e + 1] + rank[:, e:e + 1] - 1.0
    dest += oh_blk[:, e:e + 1] * seg
    poff_e = poff_e + pc[:, e:e + 1]
    ends.append(poff_e)
  dest_ref[...] = dest.astype(jnp.int32)
  blk_iota = jax.lax.broadcasted_iota(jnp.int32, (1, 128), 1).astype(_f32) * _BLK
  nfull = jnp.zeros((1, 128), jnp.int32)
  for e in range(_E):
    nfull += (blk_iota >= ends[e]).astype(jnp.int32)
  bexp_ref[...] = jnp.minimum(nfull, _E - 1)


def _route_inv_body(dest_row_ref, w_row_ref, src_ref, ws_ref):
  j = pl.program_id(0)
  slot = jax.lax.broadcasted_iota(jnp.int32, (_BLK, _S), 0) + j * _BLK
  cmp = (dest_row_ref[...] == slot).astype(_f32)
  tok = jax.lax.broadcasted_iota(jnp.int32, (_BLK, _S), 1).astype(_f32)
  matched = jnp.sum(cmp, axis=1, keepdims=True)
  srcm = jnp.sum(cmp * tok, axis=1, keepdims=True)
  spread = (jax.lax.broadcasted_iota(jnp.int32, (_BLK, 1), 0)
            + j * _BLK) % _S
  src_ref[...] = jnp.where(matched > 0.0, srcm,
                           spread.astype(_f32)).astype(jnp.int32)
  ws_ref[...] = jnp.sum(cmp * w_row_ref[...], axis=1, keepdims=True)


def _sc_gather_body(table_hbm, idx_hbm, out_hbm, idx_v, rows_v, sem):
  wid = lax.axis_index("s") * 2 + lax.axis_index("c")
  rows_per = out_hbm.shape[0] // _NSC
  base = wid * rows_per
  pltpu.sync_copy(idx_hbm.at[pl.ds(base, rows_per)], idx_v)
  pltpu.async_copy(table_hbm.at[idx_v], rows_v, sem).wait()
  pltpu.sync_copy(rows_v, out_hbm.at[pl.ds(base, rows_per)])


def _moe_mm_body(bexp_ref, xs_ref, ws_ref, wm_ref, bm_ref, ys_ref):
  ys = jnp.dot(xs_ref[...], wm_ref[0], preferred_element_type=_f32)
  ys_ref[...] = ys + ws_ref[...] * bm_ref[0].astype(_f32)


def _combine_body(x1_ref, y_ref, xout_ref):
  xout_ref[...] = x1_ref[...] + y_ref[...]


def _routed_ffn(x1, xw, wvec, eid, wm, bm):
  dest, bexp = pl.pallas_call(
      _route_prep_body,
      grid=(_S // _SB,),
      in_specs=[
          _const_spec((_S, 1)),
          pl.BlockSpec((_SB, 1), lambda i: (i, 0)),
      ],
      out_specs=[
          pl.BlockSpec((_SB, 1), lambda i: (i, 0)),
          _const_spec((1, 128)),
      ],
      out_shape=[
          jax.ShapeDtypeStruct((_S, 1), jnp.int32),
          jax.ShapeDtypeStruct((1, 128), jnp.int32),
      ],
      compiler_params=pltpu.CompilerParams(
          dimension_semantics=("arbitrary",)),
  )(eid, eid)

  srci, ws = pl.pallas_call(
      _route_inv_body,
      grid=(_NB,),
      in_specs=[
          _const_spec((1, _S)),
          _const_spec((1, _S)),
      ],
      out_specs=[
          pl.BlockSpec((_BLK, 1), lambda j: (j, 0)),
          pl.BlockSpec((_BLK, 1), lambda j: (j, 0)),
      ],
      out_shape=[
          jax.ShapeDtypeStruct((_PAD_S, 1), jnp.int32),
          jax.ShapeDtypeStruct((_PAD_S, 1), _f32),
      ],
      compiler_params=pltpu.CompilerParams(
          dimension_semantics=("arbitrary",)),
  )(dest.reshape(1, _S), wvec.reshape(1, _S))

  dest1 = dest.reshape(_S)
  mesh = plsc.VectorSubcoreMesh(core_axis_name="c", subcore_axis_name="s")
  xw_pairs = jax.lax.bitcast_convert_type(
      xw.reshape(_S, _DIM // 2, 2), jnp.int32)
  xs_pairs = pl.kernel(
      _sc_gather_body,
      out_type=jax.ShapeDtypeStruct((_PAD_S, _DIM // 2), jnp.int32),
      mesh=mesh,
      scratch_types=[
          pltpu.VMEM((_PAD_S // _NSC,), jnp.int32),
          pltpu.VMEM((_PAD_S // _NSC, _DIM // 2), jnp.int32),
          pltpu.SemaphoreType.DMA,
      ],
  )(xw_pairs, srci.reshape(_PAD_S))
  xs = jax.lax.bitcast_convert_type(xs_pairs, _bf16).reshape(_PAD_S, _DIM)

  ys = pl.pallas_call(
      _moe_mm_body,
      grid_spec=pltpu.PrefetchScalarGridSpec(
          num_scalar_prefetch=1,
          grid=(_NB,),
          in_specs=[
              pl.BlockSpec((_BLK, _DIM), lambda j, be: (j, 0)),
              pl.BlockSpec((_BLK, 1), lambda j, be: (j, 0)),
              pl.BlockSpec((1, _DIM, _DIM), lambda j, be: (be[j], 0, 0)),
              pl.BlockSpec((1, 1, _DIM), lambda j, be: (be[j], 0, 0)),
          ],
          out_specs=pl.BlockSpec((_BLK, _DIM), lambda j, be: (j, 0)),
      ),
      out_shape=jax.ShapeDtypeStruct((_PAD_S, _DIM), _f32),
      compiler_params=pltpu.CompilerParams(
          dimension_semantics=("arbitrary",)),
  )(bexp.reshape(128)[:_NB], xs, ws, wm, bm.reshape(_E, 1, _DIM))

  y = pl.kernel(
      _sc_gather_body,
      out_type=jax.ShapeDtypeStruct((_S, _DIM), _f32),
      mesh=mesh,
      scratch_types=[
          pltpu.VMEM((_S // _NSC,), jnp.int32),
          pltpu.VMEM((_S // _NSC, _DIM), _f32),
          pltpu.SemaphoreType.DMA,
      ],
  )(ys, dest1)

  return pl.pallas_call(
      _combine_body,
      grid=(_S // _SB,),
      in_specs=[
          pl.BlockSpec((_SB, _DIM), lambda i: (i, 0)),
          pl.BlockSpec((_SB, _DIM), lambda i: (i, 0)),
      ],
      out_specs=pl.BlockSpec((_SB, _DIM), lambda i: (i, 0)),
      out_shape=jax.ShapeDtypeStruct((_S, _DIM), _f32),
      compiler_params=pltpu.CompilerParams(
          dimension_semantics=("arbitrary",)),
  )(x1, y)


def _run_layer(xb, g1, b1, g2, b2, wq, wk, wsv, wso, wv, wo, wg, wm, bm):
  nblk = _S // _SB
  q, k, v, mow, moi = pl.pallas_call(
      _pre_body,
      grid=(nblk,),
      in_specs=[
          pl.BlockSpec((_SB, _DIM), lambda i: (i, 0)),
          _const_spec((1, _DIM)), _const_spec((1, _DIM)),
          _const_spec((_DIM, _DIM)), _const_spec((_DIM, _DIM)),
          _const_spec((_DIM, _E * _H)), _const_spec((_DIM, _E * _H)),
          _const_spec((_E, _DIM, _DIM)),
      ],
      out_specs=[
          pl.BlockSpec((_SB, _DIM), lambda i: (i, 0)),
          pl.BlockSpec((_SB, _DIM), lambda i: (i, 0)),
          pl.BlockSpec((_SB, _DIM), lambda i: (i, 0)),
          pl.BlockSpec((_SB, _H), lambda i: (i, 0)),
          pl.BlockSpec((_SB, _H), lambda i: (i, 0)),
      ],
      out_shape=[
          jax.ShapeDtypeStruct((_S, _DIM), _bf16),
          jax.ShapeDtypeStruct((_S, _DIM), _bf16),
          jax.ShapeDtypeStruct((_S, _DIM), _bf16),
          jax.ShapeDtypeStruct((_S, _H), _f32),
          jax.ShapeDtypeStruct((_S, _H), jnp.int32),
      ],
      compiler_params=pltpu.CompilerParams(
          dimension_semantics=("arbitrary",)),
  )(xb, g1, b1, wq, wk, wsv, wso, wv)

  o = pl.pallas_call(
      _attn_body,
      grid=(_S // _SQ,),
      in_specs=[
          pl.BlockSpec((_SQ, _DIM), lambda i: (i, 0)),
          _const_spec((_S, _DIM)),
          _const_spec((_S, _DIM)),
      ],
      out_specs=pl.BlockSpec((_SQ, _DIM), lambda i: (i, 0)),
      out_shape=jax.ShapeDtypeStruct((_S, _DIM), _bf16),
      compiler_params=pltpu.CompilerParams(
          dimension_semantics=("arbitrary",)),
  )(q, k, v)

  x1, xw, wvec, eid = pl.pallas_call(
      _post_body,
      grid=(nblk,),
      in_specs=[
          pl.BlockSpec((_SB, _DIM), lambda i: (i, 0)),
          pl.BlockSpec((_SB, _DIM), lambda i: (i, 0)),
          pl.BlockSpec((_SB, _H), lambda i: (i, 0)),
          pl.BlockSpec((_SB, _H), lambda i: (i, 0)),
          _const_spec((_E, _DIM, _DIM)),
          _const_spec((1, _DIM)), _const_spec((1, _DIM)),
          _const_spec((_DIM, _E)),
      ],
      out_specs=[
          pl.BlockSpec((_SB, _DIM), lambda i: (i, 0)),
          pl.BlockSpec((_SB, _DIM), lambda i: (i, 0)),
          pl.BlockSpec((_SB, 1), lambda i: (i, 0)),
          pl.BlockSpec((_SB, 1), lambda i: (i, 0)),
      ],
      out_shape=[
          jax.ShapeDtypeStruct((_S, _DIM), _f32),
          jax.ShapeDtypeStruct((_S, _DIM), _bf16),
          jax.ShapeDtypeStruct((_S, 1), _f32),
          jax.ShapeDtypeStruct((_S, 1), jnp.int32),
      ],
      compiler_params=pltpu.CompilerParams(
          dimension_semantics=("arbitrary",)),
  )(xb, o, mow, moi, wo, g2, b2, wg)

  return _routed_ffn(x1, xw, wvec, eid, wm, bm)


def kernel(x, ln1_g, ln1_b, ln2_g, ln2_b, Wq, Wk, Wv, Wo, Wsv, Wso, Wg, Wm, bm):
  xb = x[0]
  for l in range(2):
    xb = _run_layer(
        xb,
        ln1_g[l][None, :], ln1_b[l][None, :],
        ln2_g[l][None, :], ln2_b[l][None, :],
        Wq[l].astype(_bf16), Wk[l].astype(_bf16),
        Wsv[l].transpose(0, 2, 1).reshape(_DIM, _E * _H).astype(_bf16),
        Wso[l].transpose(0, 2, 1).reshape(_DIM, _E * _H).astype(_bf16),
        Wv[l].reshape(_E, _DIM, _DIM).astype(_bf16),
        Wo[l].reshape(_E, _DIM, _DIM).astype(_bf16),
        Wg[l].astype(_bf16),
        Wm[l].astype(_bf16),
        bm[l].astype(_bf16),
    )
  return xb[None]


# speedup vs baseline: 1.2308x; 1.2308x over previous
import functools

import jax
import jax.numpy as jnp
from jax import lax
from jax.experimental import pallas as pl
from jax.experimental.pallas import tpu as pltpu
from jax.experimental.pallas import tpu_sc as plsc

_DIM = 768
_H = 12
_DH = 64
_E = 6
_S = 2048

_SB = 256
_SQ = 512

_bf16 = jnp.bfloat16
_f32 = jnp.float32


def _head_expand_matrix():
  rows = jax.lax.broadcasted_iota(jnp.int32, (_H, _DIM), 0)
  cols = jax.lax.broadcasted_iota(jnp.int32, (_H, _DIM), 1)
  return (cols // _DH == rows).astype(_f32)


def _top1_sigmoid(scores_em):
  best_s = jnp.full((scores_em.shape[0], _H), -jnp.inf, _f32)
  best_i = jnp.zeros((scores_em.shape[0], _H), jnp.int32)
  for e in range(_E):
    se = scores_em[:, e * _H:(e + 1) * _H]
    upd = se > best_s
    best_s = jnp.where(upd, se, best_s)
    best_i = jnp.where(upd, e, best_i)
  return jax.nn.sigmoid(best_s), best_i


def _layernorm(xb, g, b):
  m = jnp.mean(xb, axis=-1, keepdims=True)
  xc = xb - m
  v = jnp.mean(xc * xc, axis=-1, keepdims=True)
  return xc / jnp.sqrt(v + 1e-5) * g + b


def _pre_body(x_ref, g_ref, b_ref, wq_ref, wk_ref, wsv_ref, wso_ref, wv_ref,
              q_ref, k_ref, v_ref, mow_ref, moi_ref):
  xn = _layernorm(x_ref[...], g_ref[...], b_ref[...])
  xnb = xn.astype(_bf16)
  q_ref[...] = jnp.dot(xnb, wq_ref[...], preferred_element_type=_f32).astype(_bf16)
  k_ref[...] = jnp.dot(xnb, wk_ref[...], preferred_element_type=_f32).astype(_bf16)
  sv = jnp.dot(xnb, wsv_ref[...], preferred_element_type=_f32)
  so = jnp.dot(xnb, wso_ref[...], preferred_element_type=_f32)
  vw, vi = _top1_sigmoid(sv)
  ow, oi = _top1_sigmoid(so)
  mow_ref[...] = ow
  moi_ref[...] = oi
  rexp = _head_expand_matrix()
  vw_b = vw.astype(_bf16).astype(_f32)
  vacc = jnp.zeros((x_ref.shape[0], _DIM), _f32)
  for e in range(_E):
    ve = jnp.dot(xnb, wv_ref[e], preferred_element_type=_f32)
    ve_b = ve.astype(_bf16).astype(_f32)
    m768 = jnp.dot(jnp.where(vi == e, vw_b, 0.0), rexp,
                   preferred_element_type=_f32)
    vacc += m768 * ve_b
  v_ref[...] = vacc.astype(_bf16)


def _attn_body(q_ref, k_ref, v_ref, o_ref):
  scale = 1.0 / jnp.sqrt(jnp.float32(_DH))
  for h in range(_H):
    sl = slice(h * _DH, (h + 1) * _DH)
    qh = q_ref[:, sl]
    kh = k_ref[:, sl]
    vh = v_ref[:, sl]
    s = jax.lax.dot_general(qh, kh, (((1,), (1,)), ((), ())),
                            preferred_element_type=_f32) * scale
    m = jnp.max(s, axis=-1, keepdims=True)
    p = jnp.exp(s - m)
    l = jnp.sum(p, axis=-1, keepdims=True)
    pn = (p * (1.0 / l)).astype(_bf16)
    o = jax.lax.dot_general(pn, vh, (((1,), (0,)), ((), ())),
                            preferred_element_type=_f32)
    o_ref[:, sl] = o.astype(_bf16)


def _post_body(x_ref, o_ref, mow_ref, moi_ref, wo_ref, g2_ref, b2_ref,
               wg_ref, x1_ref, xw_ref, wv_out_ref, eid_ref):
  rexp = _head_expand_matrix()
  of = o_ref[...].astype(_f32)
  ow_b = mow_ref[...].astype(_bf16).astype(_f32)
  oi = moi_ref[...]
  acc = jnp.zeros((x_ref.shape[0], _DIM), _f32)
  for e in range(_E):
    m768 = jnp.dot(jnp.where(oi == e, ow_b, 0.0), rexp,
                   preferred_element_type=_f32)
    oe = (of * m768).astype(_bf16)
    acc += jnp.dot(oe, wo_ref[e], preferred_element_type=_f32)
  x1 = x_ref[...] + acc
  xn2 = _layernorm(x1, g2_ref[...], b2_ref[...])
  xn2b = xn2.astype(_bf16)
  logits = jnp.dot(xn2b, wg_ref[...], preferred_element_type=_f32)
  lm = jnp.max(logits, axis=-1, keepdims=True)
  ex = jnp.exp(logits - lm)
  gw = 1.0 / jnp.sum(ex, axis=-1, keepdims=True)
  gb = jnp.full((x_ref.shape[0], 1), -jnp.inf, _f32)
  gi = jnp.zeros((x_ref.shape[0], 1), jnp.int32)
  for e in range(_E):
    ge = logits[:, e:e + 1]
    upd = ge > gb
    gb = jnp.where(upd, ge, gb)
    gi = jnp.where(upd, e, gi)
  xn2_b32 = xn2b.astype(_f32)
  gw_b = gw.astype(_bf16).astype(_f32)
  x1_ref[...] = x1
  xw_ref[...] = xn2_b32 * gw_b
  wv_out_ref[...] = gw_b
  eid_ref[...] = gi


def _const_spec(shape):
  nd = len(shape)
  return pl.BlockSpec(shape, lambda *_: (0,) * nd)


_BLK = 256
_PAD_S = _S + _E * _BLK
_NB = _PAD_S // _BLK
_NSC = 32


def _route_prep_body(eid_full_ref, eid_blk_ref, dest_ref, bexp_ref):
  i = pl.program_id(0)
  e_iota_full = jax.lax.broadcasted_iota(jnp.int32, (_S, _E), 1)
  oh_full = (eid_full_ref[...] == e_iota_full).astype(_f32)
  counts = jnp.sum(oh_full, axis=0, keepdims=True)
  pc = jnp.floor((counts + (_BLK - 1)) / _BLK) * _BLK
  row_iota = jax.lax.broadcasted_iota(jnp.int32, (_S, _E), 0)
  prior = jnp.sum(jnp.where(row_iota < i * _SB, oh_full, 0.0),
                  axis=0, keepdims=True)
  e_iota_blk = jax.lax.broadcasted_iota(jnp.int32, (_SB, _E), 1)
  oh_blk = (eid_blk_ref[...] == e_iota_blk).astype(_f32)
  lt_r = jax.lax.broadcasted_iota(jnp.int32, (_SB, _SB), 0)
  lt_c = jax.lax.broadcasted_iota(jnp.int32, (_SB, _SB), 1)
  lt = (lt_c <= lt_r).astype(_f32)
  rank = jnp.dot(lt, oh_blk, preferred_element_type=_f32,
                 precision=jax.lax.Precision.HIGHEST)
  dest = jnp.zeros((_SB, 1), _f32)
  poff_e = jnp.zeros((1, 1), _f32)
  ends = []
  for e in range(_E):
    seg = poff_e + prior[:, e:e + 1] + rank[:, e:e + 1] - 1.0
    dest += oh_blk[:, e:e + 1] * seg
    poff_e = poff_e + pc[:, e:e + 1]
    ends.append(poff_e)
  dest_ref[...] = dest.astype(jnp.int32)
  blk_iota = jax.lax.broadcasted_iota(jnp.int32, (1, 128), 1).astype(_f32) * _BLK
  nfull = jnp.zeros((1, 128), jnp.int32)
  for e in range(_E):
    nfull += (blk_iota >= ends[e]).astype(jnp.int32)
  bexp_ref[...] = jnp.minimum(nfull, _E - 1)


def _route_inv_body(dest_row_ref, w_row_ref, src_ref, ws_ref):
  j = pl.program_id(0)
  slot = jax.lax.broadcasted_iota(jnp.int32, (_BLK, _S), 0) + j * _BLK
  cmp = (dest_row_ref[...] == slot).astype(_f32)
  tok = jax.lax.broadcasted_iota(jnp.int32, (1, _S), 1).astype(_f32)
  one = jnp.ones((1, _S), _f32)
  tw = jnp.concatenate([tok, w_row_ref[...], one], axis=0)
  red = jax.lax.dot_general(cmp, tw, (((1,), (1,)), ((), ())),
                            preferred_element_type=_f32,
                            precision=jax.lax.Precision.HIGHEST)
  spread = (jax.lax.broadcasted_iota(jnp.int32, (_BLK, 1), 0)
            + j * _BLK) % _S
  src_ref[...] = jnp.where(red[:, 2:3] > 0.0, red[:, 0:1],
                           spread.astype(_f32)).astype(jnp.int32)
  ws_ref[...] = red[:, 1:2]


def _sc_gather_body(table_hbm, idx_hbm, out_hbm, idx_v, rows_v, sem):
  wid = lax.axis_index("s") * 2 + lax.axis_index("c")
  rows_per = out_hbm.shape[0] // _NSC
  base = wid * rows_per
  pltpu.sync_copy(idx_hbm.at[pl.ds(base, rows_per)], idx_v)
  pltpu.async_copy(table_hbm.at[idx_v], rows_v, sem).wait()
  pltpu.sync_copy(rows_v, out_hbm.at[pl.ds(base, rows_per)])


def _moe_mm_body(bexp_ref, xs_ref, ws_ref, wm_ref, bm_ref, ys_ref):
  ys = jnp.dot(xs_ref[...].astype(_bf16), wm_ref[0], preferred_element_type=_f32)
  ys_ref[...] = ys + ws_ref[...] * bm_ref[0].astype(_f32)


def _combine_body(x1_ref, y_ref, xout_ref):
  xout_ref[...] = x1_ref[...] + y_ref[...]


def _routed_ffn(x1, xw, wvec, eid, wm, bm):
  dest, bexp = pl.pallas_call(
      _route_prep_body,
      grid=(_S // _SB,),
      in_specs=[
          _const_spec((_S, 1)),
          pl.BlockSpec((_SB, 1), lambda i: (i, 0)),
      ],
      out_specs=[
          pl.BlockSpec((_SB, 1), lambda i: (i, 0)),
          _const_spec((1, 128)),
      ],
      out_shape=[
          jax.ShapeDtypeStruct((_S, 1), jnp.int32),
          jax.ShapeDtypeStruct((1, 128), jnp.int32),
      ],
      compiler_params=pltpu.CompilerParams(
          dimension_semantics=("arbitrary",)),
  )(eid, eid)

  srci, ws = pl.pallas_call(
      _route_inv_body,
      grid=(_NB,),
      in_specs=[
          _const_spec((1, _S)),
          _const_spec((1, _S)),
      ],
      out_specs=[
          pl.BlockSpec((_BLK, 1), lambda j: (j, 0)),
          pl.BlockSpec((_BLK, 1), lambda j: (j, 0)),
      ],
      out_shape=[
          jax.ShapeDtypeStruct((_PAD_S, 1), jnp.int32),
          jax.ShapeDtypeStruct((_PAD_S, 1), _f32),
      ],
      compiler_params=pltpu.CompilerParams(
          dimension_semantics=("arbitrary",)),
  )(dest.reshape(1, _S), wvec.reshape(1, _S))

  dest1 = dest.reshape(_S)
  mesh = plsc.VectorSubcoreMesh(core_axis_name="c", subcore_axis_name="s")
  xs = pl.kernel(
      _sc_gather_body,
      out_type=jax.ShapeDtypeStruct((_PAD_S, _DIM), _f32),
      mesh=mesh,
      scratch_types=[
          pltpu.VMEM((_PAD_S // _NSC,), jnp.int32),
          pltpu.VMEM((_PAD_S // _NSC, _DIM), _f32),
          pltpu.SemaphoreType.DMA,
      ],
  )(xw, srci.reshape(_PAD_S))

  ys = pl.pallas_call(
      _moe_mm_body,
      grid_spec=pltpu.PrefetchScalarGridSpec(
          num_scalar_prefetch=1,
          grid=(_NB,),
          in_specs=[
              pl.BlockSpec((_BLK, _DIM), lambda j, be: (j, 0)),
              pl.BlockSpec((_BLK, 1), lambda j, be: (j, 0)),
              pl.BlockSpec((1, _DIM, _DIM), lambda j, be: (be[j], 0, 0)),
              pl.BlockSpec((1, 1, _DIM), lambda j, be: (be[j], 0, 0)),
          ],
          out_specs=pl.BlockSpec((_BLK, _DIM), lambda j, be: (j, 0)),
      ),
      out_shape=jax.ShapeDtypeStruct((_PAD_S, _DIM), _f32),
      compiler_params=pltpu.CompilerParams(
          dimension_semantics=("arbitrary",)),
  )(bexp.reshape(128)[:_NB], xs, ws, wm, bm.reshape(_E, 1, _DIM))

  y = pl.kernel(
      _sc_gather_body,
      out_type=jax.ShapeDtypeStruct((_S, _DIM), _f32),
      mesh=mesh,
      scratch_types=[
          pltpu.VMEM((_S // _NSC,), jnp.int32),
          pltpu.VMEM((_S // _NSC, _DIM), _f32),
          pltpu.SemaphoreType.DMA,
      ],
  )(ys, dest1)

  return pl.pallas_call(
      _combine_body,
      grid=(_S // _SB,),
      in_specs=[
          pl.BlockSpec((_SB, _DIM), lambda i: (i, 0)),
          pl.BlockSpec((_SB, _DIM), lambda i: (i, 0)),
      ],
      out_specs=pl.BlockSpec((_SB, _DIM), lambda i: (i, 0)),
      out_shape=jax.ShapeDtypeStruct((_S, _DIM), _f32),
      compiler_params=pltpu.CompilerParams(
          dimension_semantics=("arbitrary",)),
  )(x1, y)


def _run_layer(xb, g1, b1, g2, b2, wq, wk, wsv, wso, wv, wo, wg, wm, bm):
  nblk = _S // _SB
  q, k, v, mow, moi = pl.pallas_call(
      _pre_body,
      grid=(nblk,),
      in_specs=[
          pl.BlockSpec((_SB, _DIM), lambda i: (i, 0)),
          _const_spec((1, _DIM)), _const_spec((1, _DIM)),
          _const_spec((_DIM, _DIM)), _const_spec((_DIM, _DIM)),
          _const_spec((_DIM, _E * _H)), _const_spec((_DIM, _E * _H)),
          _const_spec((_E, _DIM, _DIM)),
      ],
      out_specs=[
          pl.BlockSpec((_SB, _DIM), lambda i: (i, 0)),
          pl.BlockSpec((_SB, _DIM), lambda i: (i, 0)),
          pl.BlockSpec((_SB, _DIM), lambda i: (i, 0)),
          pl.BlockSpec((_SB, _H), lambda i: (i, 0)),
          pl.BlockSpec((_SB, _H), lambda i: (i, 0)),
      ],
      out_shape=[
          jax.ShapeDtypeStruct((_S, _DIM), _bf16),
          jax.ShapeDtypeStruct((_S, _DIM), _bf16),
          jax.ShapeDtypeStruct((_S, _DIM), _bf16),
          jax.ShapeDtypeStruct((_S, _H), _f32),
          jax.ShapeDtypeStruct((_S, _H), jnp.int32),
      ],
      compiler_params=pltpu.CompilerParams(
          dimension_semantics=("arbitrary",)),
  )(xb, g1, b1, wq, wk, wsv, wso, wv)

  o = pl.pallas_call(
      _attn_body,
      grid=(_S // _SQ,),
      in_specs=[
          pl.BlockSpec((_SQ, _DIM), lambda i: (i, 0)),
          _const_spec((_S, _DIM)),
          _const_spec((_S, _DIM)),
      ],
      out_specs=pl.BlockSpec((_SQ, _DIM), lambda i: (i, 0)),
      out_shape=jax.ShapeDtypeStruct((_S, _DIM), _bf16),
      compiler_params=pltpu.CompilerParams(
          dimension_semantics=("arbitrary",)),
  )(q, k, v)

  x1, xw, wvec, eid = pl.pallas_call(
      _post_body,
      grid=(nblk,),
      in_specs=[
          pl.BlockSpec((_SB, _DIM), lambda i: (i, 0)),
          pl.BlockSpec((_SB, _DIM), lambda i: (i, 0)),
          pl.BlockSpec((_SB, _H), lambda i: (i, 0)),
          pl.BlockSpec((_SB, _H), lambda i: (i, 0)),
          _const_spec((_E, _DIM, _DIM)),
          _const_spec((1, _DIM)), _const_spec((1, _DIM)),
          _const_spec((_DIM, _E)),
      ],
      out_specs=[
          pl.BlockSpec((_SB, _DIM), lambda i: (i, 0)),
          pl.BlockSpec((_SB, _DIM), lambda i: (i, 0)),
          pl.BlockSpec((_SB, 1), lambda i: (i, 0)),
          pl.BlockSpec((_SB, 1), lambda i: (i, 0)),
      ],
      out_shape=[
          jax.ShapeDtypeStruct((_S, _DIM), _f32),
          jax.ShapeDtypeStruct((_S, _DIM), _f32),
          jax.ShapeDtypeStruct((_S, 1), _f32),
          jax.ShapeDtypeStruct((_S, 1), jnp.int32),
      ],
      compiler_params=pltpu.CompilerParams(
          dimension_semantics=("arbitrary",)),
  )(xb, o, mow, moi, wo, g2, b2, wg)

  return _routed_ffn(x1, xw, wvec, eid, wm, bm)


def kernel(x, ln1_g, ln1_b, ln2_g, ln2_b, Wq, Wk, Wv, Wo, Wsv, Wso, Wg, Wm, bm):
  xb = x[0]
  for l in range(2):
    xb = _run_layer(
        xb,
        ln1_g[l][None, :], ln1_b[l][None, :],
        ln2_g[l][None, :], ln2_b[l][None, :],
        Wq[l].astype(_bf16), Wk[l].astype(_bf16),
        Wsv[l].transpose(0, 2, 1).reshape(_DIM, _E * _H).astype(_bf16),
        Wso[l].transpose(0, 2, 1).reshape(_DIM, _E * _H).astype(_bf16),
        Wv[l].reshape(_E, _DIM, _DIM).astype(_bf16),
        Wo[l].reshape(_E, _DIM, _DIM).astype(_bf16),
        Wg[l].astype(_bf16),
        Wm[l].astype(_bf16),
        bm[l].astype(_bf16),
    )
  return xb[None]


# speedup vs baseline: 1.6534x; 1.3434x over previous
import jax
import jax.numpy as jnp
from jax.experimental import pallas as pl
from jax.experimental.pallas import tpu as pltpu

_DIM = 768
_H = 12
_DH = 64
_E = 6
_S = 2048

_SB = 256
_SQ = 512

_bf16 = jnp.bfloat16
_f32 = jnp.float32


def _head_expand_matrix():
  rows = jax.lax.broadcasted_iota(jnp.int32, (_H, _DIM), 0)
  cols = jax.lax.broadcasted_iota(jnp.int32, (_H, _DIM), 1)
  return (cols // _DH == rows).astype(_f32)


def _top1_sigmoid(scores_em):
  best_s = jnp.full((scores_em.shape[0], _H), -jnp.inf, _f32)
  best_i = jnp.zeros((scores_em.shape[0], _H), jnp.int32)
  for e in range(_E):
    se = scores_em[:, e * _H:(e + 1) * _H]
    upd = se > best_s
    best_s = jnp.where(upd, se, best_s)
    best_i = jnp.where(upd, e, best_i)
  return jax.nn.sigmoid(best_s), best_i


def _layernorm(xb, g, b):
  m = jnp.mean(xb, axis=-1, keepdims=True)
  xc = xb - m
  v = jnp.mean(xc * xc, axis=-1, keepdims=True)
  return xc / jnp.sqrt(v + 1e-5) * g + b


def _pre_body(x_ref, g_ref, b_ref, wq_ref, wk_ref, wsv_ref, wso_ref, wv_ref,
              q_ref, k_ref, v_ref, mow_ref, moi_ref):
  xn = _layernorm(x_ref[...], g_ref[...], b_ref[...])
  xnb = xn.astype(_bf16)
  q_ref[...] = jnp.dot(xnb, wq_ref[...], preferred_element_type=_f32).astype(_bf16)
  k_ref[...] = jnp.dot(xnb, wk_ref[...], preferred_element_type=_f32).astype(_bf16)
  sv = jnp.dot(xnb, wsv_ref[...], preferred_element_type=_f32)
  so = jnp.dot(xnb, wso_ref[...], preferred_element_type=_f32)
  vw, vi = _top1_sigmoid(sv)
  ow, oi = _top1_sigmoid(so)
  mow_ref[...] = ow
  moi_ref[...] = oi
  rexp = _head_expand_matrix()
  vw_b = vw.astype(_bf16).astype(_f32)
  vacc = jnp.zeros((x_ref.shape[0], _DIM), _f32)
  for e in range(_E):
    ve = jnp.dot(xnb, wv_ref[e], preferred_element_type=_f32)
    ve_b = ve.astype(_bf16).astype(_f32)
    m768 = jnp.dot(jnp.where(vi == e, vw_b, 0.0), rexp,
                   preferred_element_type=_f32)
    vacc += m768 * ve_b
  v_ref[...] = vacc.astype(_bf16)


def _attn_body(q_ref, k_ref, v_ref, o_ref):
  scale = 1.0 / jnp.sqrt(jnp.float32(_DH))
  for h in range(_H):
    sl = slice(h * _DH, (h + 1) * _DH)
    qh = q_ref[:, sl]
    kh = k_ref[:, sl]
    vh = v_ref[:, sl]
    s = jax.lax.dot_general(qh, kh, (((1,), (1,)), ((), ())),
                            preferred_element_type=_f32) * scale
    m = jnp.max(s, axis=-1, keepdims=True)
    p = jnp.exp(s - m)
    l = jnp.sum(p, axis=-1, keepdims=True)
    pn = (p * (1.0 / l)).astype(_bf16)
    o = jax.lax.dot_general(pn, vh, (((1,), (0,)), ((), ())),
                            preferred_element_type=_f32)
    o_ref[:, sl] = o.astype(_bf16)


def _post_body(x_ref, o_ref, mow_ref, moi_ref, wo_ref, g2_ref, b2_ref,
               wg_ref, wm_ref, bm_ref, xout_ref):
  rexp = _head_expand_matrix()
  of = o_ref[...].astype(_f32)
  ow_b = mow_ref[...].astype(_bf16).astype(_f32)
  oi = moi_ref[...]
  acc = jnp.zeros((x_ref.shape[0], _DIM), _f32)
  for e in range(_E):
    m768 = jnp.dot(jnp.where(oi == e, ow_b, 0.0), rexp,
                   preferred_element_type=_f32)
    oe = (of * m768).astype(_bf16)
    acc += jnp.dot(oe, wo_ref[e], preferred_element_type=_f32)
  x1 = x_ref[...] + acc
  xn2 = _layernorm(x1, g2_ref[...], b2_ref[...])
  xn2b = xn2.astype(_bf16)
  logits = jnp.dot(xn2b, wg_ref[...], preferred_element_type=_f32)
  lm = jnp.max(logits, axis=-1, keepdims=True)
  ex = jnp.exp(logits - lm)
  gw = 1.0 / jnp.sum(ex, axis=-1, keepdims=True)
  gb = jnp.full((x_ref.shape[0], 1), -jnp.inf, _f32)
  gi = jnp.zeros((x_ref.shape[0], 1), jnp.int32)
  for e in range(_E):
    ge = logits[:, e:e + 1]
    upd = ge > gb
    gb = jnp.where(upd, ge, gb)
    gi = jnp.where(upd, e, gi)
  xn2_b32 = xn2b.astype(_f32)
  gw_b = gw.astype(_bf16).astype(_f32)
  bmf = bm_ref[...].astype(_f32)
  facc = jnp.zeros((x_ref.shape[0], _DIM), _f32)
  for e in range(_E):
    w_e = jnp.where(gi == e, gw_b, 0.0)
    fin = (xn2_b32 * w_e).astype(_bf16)
    fe = jnp.dot(fin, wm_ref[e], preferred_element_type=_f32)
    facc += fe + w_e * bmf[e:e + 1, :]
  xout_ref[...] = x1 + facc


def _const_spec(shape):
  nd = len(shape)
  return pl.BlockSpec(shape, lambda *_: (0,) * nd)


def _run_layer(xb, g1, b1, g2, b2, wq, wk, wsv, wso, wv, wo, wg, wm, bm):
  nblk = _S // _SB
  q, k, v, mow, moi = pl.pallas_call(
      _pre_body,
      grid=(nblk,),
      in_specs=[
          pl.BlockSpec((_SB, _DIM), lambda i: (i, 0)),
          _const_spec((1, _DIM)), _const_spec((1, _DIM)),
          _const_spec((_DIM, _DIM)), _const_spec((_DIM, _DIM)),
          _const_spec((_DIM, _E * _H)), _const_spec((_DIM, _E * _H)),
          _const_spec((_E, _DIM, _DIM)),
      ],
      out_specs=[
          pl.BlockSpec((_SB, _DIM), lambda i: (i, 0)),
          pl.BlockSpec((_SB, _DIM), lambda i: (i, 0)),
          pl.BlockSpec((_SB, _DIM), lambda i: (i, 0)),
          pl.BlockSpec((_SB, _H), lambda i: (i, 0)),
          pl.BlockSpec((_SB, _H), lambda i: (i, 0)),
      ],
      out_shape=[
          jax.ShapeDtypeStruct((_S, _DIM), _bf16),
          jax.ShapeDtypeStruct((_S, _DIM), _bf16),
          jax.ShapeDtypeStruct((_S, _DIM), _bf16),
          jax.ShapeDtypeStruct((_S, _H), _f32),
          jax.ShapeDtypeStruct((_S, _H), jnp.int32),
      ],
      compiler_params=pltpu.CompilerParams(
          dimension_semantics=("arbitrary",)),
  )(xb, g1, b1, wq, wk, wsv, wso, wv)

  o = pl.pallas_call(
      _attn_body,
      grid=(_S // _SQ,),
      in_specs=[
          pl.BlockSpec((_SQ, _DIM), lambda i: (i, 0)),
          _const_spec((_S, _DIM)),
          _const_spec((_S, _DIM)),
      ],
      out_specs=pl.BlockSpec((_SQ, _DIM), lambda i: (i, 0)),
      out_shape=jax.ShapeDtypeStruct((_S, _DIM), _bf16),
      compiler_params=pltpu.CompilerParams(
          dimension_semantics=("arbitrary",)),
  )(q, k, v)

  xout = pl.pallas_call(
      _post_body,
      grid=(nblk,),
      in_specs=[
          pl.BlockSpec((_SB, _DIM), lambda i: (i, 0)),
          pl.BlockSpec((_SB, _DIM), lambda i: (i, 0)),
          pl.BlockSpec((_SB, _H), lambda i: (i, 0)),
          pl.BlockSpec((_SB, _H), lambda i: (i, 0)),
          _const_spec((_E, _DIM, _DIM)),
          _const_spec((1, _DIM)), _const_spec((1, _DIM)),
          _const_spec((_DIM, _E)),
          _const_spec((_E, _DIM, _DIM)),
          _const_spec((_E, _DIM)),
      ],
      out_specs=pl.BlockSpec((_SB, _DIM), lambda i: (i, 0)),
      out_shape=jax.ShapeDtypeStruct((_S, _DIM), _f32),
      compiler_params=pltpu.CompilerParams(
          dimension_semantics=("arbitrary",)),
  )(xb, o, mow, moi, wo, g2, b2, wg, wm, bm)
  return xout


def kernel(x, ln1_g, ln1_b, ln2_g, ln2_b, Wq, Wk, Wv, Wo, Wsv, Wso, Wg, Wm, bm):
  xb = x[0]
  for l in range(2):
    xb = _run_layer(
        xb,
        ln1_g[l][None, :], ln1_b[l][None, :],
        ln2_g[l][None, :], ln2_b[l][None, :],
        Wq[l].astype(_bf16), Wk[l].astype(_bf16),
        Wsv[l].transpose(0, 2, 1).reshape(_DIM, _E * _H).astype(_bf16),
        Wso[l].transpose(0, 2, 1).reshape(_DIM, _E * _H).astype(_bf16),
        Wv[l].reshape(_E, _DIM, _DIM).astype(_bf16),
        Wo[l].reshape(_E, _DIM, _DIM).astype(_bf16),
        Wg[l].astype(_bf16),
        Wm[l].astype(_bf16),
        bm[l].astype(_bf16),
    )
  return xb[None]
